# Initial kernel scaffold; baseline (speedup 1.0000x reference)
import jax
import jax.numpy as jnp
from jax.experimental import pallas as pl


def kernel(x, pe_table):
    raise NotImplementedError("write your pallas kernel here")


# speedup vs baseline: 1.4758x; 1.4758x over previous
import functools

import jax
import jax.numpy as jnp
from jax.experimental import pallas as pl
from jax.experimental.pallas import tpu as pltpu


def _add_kernel(x_ref, pe_ref, o_ref):
    o_ref[...] = x_ref[...] + pe_ref[...]


@jax.jit
def kernel(x, pe_table):
    B, S, D = x.shape
    R = 256

    grid = (S // R, B)

    return pl.pallas_call(
        _add_kernel,
        grid=grid,
        in_specs=[
            pl.BlockSpec((1, R, D), lambda i, j: (j, i, 0)),
            pl.BlockSpec((R, D), lambda i, j: (i, 0)),
        ],
        out_specs=pl.BlockSpec((1, R, D), lambda i, j: (j, i, 0)),
        out_shape=jax.ShapeDtypeStruct((B, S, D), x.dtype),
        compiler_params=pltpu.CompilerParams(
            dimension_semantics=("arbitrary", "arbitrary"),
        ),
    )(x, pe_table)


# speedup vs baseline: 1.9341x; 1.3105x over previous
import functools

import jax
import jax.numpy as jnp
from jax.experimental import pallas as pl
from jax.experimental.pallas import tpu as pltpu


def _add_kernel(x_ref, pe_ref, o_ref):
    o_ref[...] = x_ref[...] + pe_ref[...]


@jax.jit
def kernel(x, pe_table):
    B, S, D = x.shape
    R = 512

    grid = (S // R, B)

    return pl.pallas_call(
        _add_kernel,
        grid=grid,
        in_specs=[
            pl.BlockSpec((1, R, D), lambda i, j: (j, i, 0)),
            pl.BlockSpec((R, D), lambda i, j: (i, 0)),
        ],
        out_specs=pl.BlockSpec((1, R, D), lambda i, j: (j, i, 0)),
        out_shape=jax.ShapeDtypeStruct((B, S, D), x.dtype),
        compiler_params=pltpu.CompilerParams(
            dimension_semantics=("arbitrary", "arbitrary"),
        ),
    )(x, pe_table)


# speedup vs baseline: 2.1097x; 1.0908x over previous
import functools

import jax
import jax.numpy as jnp
from jax.experimental import pallas as pl
from jax.experimental.pallas import tpu as pltpu


def _add_kernel(x_ref, pe_ref, o_ref):
    o_ref[...] = x_ref[...] + pe_ref[...]


@jax.jit
def kernel(x, pe_table):
    B, S, D = x.shape
    R = 1024

    grid = (S // R, B)

    return pl.pallas_call(
        _add_kernel,
        grid=grid,
        in_specs=[
            pl.BlockSpec((1, R, D), lambda i, j: (j, i, 0)),
            pl.BlockSpec((R, D), lambda i, j: (i, 0)),
        ],
        out_specs=pl.BlockSpec((1, R, D), lambda i, j: (j, i, 0)),
        out_shape=jax.ShapeDtypeStruct((B, S, D), x.dtype),
        compiler_params=pltpu.CompilerParams(
            dimension_semantics=("arbitrary", "arbitrary"),
        ),
    )(x, pe_table)


# speedup vs baseline: 2.2887x; 1.0849x over previous
import functools

import jax
import jax.numpy as jnp
from jax.experimental import pallas as pl
from jax.experimental.pallas import tpu as pltpu


def _add_kernel(x_ref, pe_ref, o_ref):
    o_ref[...] = x_ref[...] + pe_ref[...]


@jax.jit
def kernel(x, pe_table):
    B, S, D = x.shape
    R = 2048

    grid = (S // R, B)

    return pl.pallas_call(
        _add_kernel,
        grid=grid,
        in_specs=[
            pl.BlockSpec((1, R, D), lambda i, j: (j, i, 0)),
            pl.BlockSpec((R, D), lambda i, j: (i, 0)),
        ],
        out_specs=pl.BlockSpec((1, R, D), lambda i, j: (j, i, 0)),
        out_shape=jax.ShapeDtypeStruct((B, S, D), x.dtype),
        compiler_params=pltpu.CompilerParams(
            dimension_semantics=("arbitrary", "arbitrary"),
        ),
    )(x, pe_table)
